# baseline (speedup 1.0000x reference)
import functools

import jax
import jax.numpy as jnp
from jax import lax
from jax.experimental import pallas as pl
from jax.experimental.pallas import tpu as pltpu
from jax.experimental.pallas import tpu_sc as plsc

N_NODES = 10000
N_PAD = 10240
N_EDGES = 160000
E_PAD = 163840
HID = 128
EF = 16
NC = 2
NS = 16
NW = NC * NS
LANES = 16
NSLICE = N_PAD // NS
PADROWS = (E_PAD - N_EDGES) // NW

EDGES_HI = 5008
EDGES_LO = 4992
CHUNK = 96
NCHUNK = 52
NBLK_L = 40
EBLK_L = E_PAD // NBLK_L

_MESH = plsc.VectorSubcoreMesh(
    core_axis_name="c", subcore_axis_name="s", num_cores=NC, num_subcores=NS)


def _worker_ids():
    cid = lax.axis_index("c")
    sid = lax.axis_index("s")
    wid = sid * NC + cid
    base = jnp.where(wid < 16, wid * EDGES_HI,
                     16 * EDGES_HI + (wid - 16) * EDGES_LO)
    return cid, sid, wid, base


def _sc_gather_body(s_hbm, d_hbm, ea_hbm, row_hbm, col_hbm, z2_hbm,
                    g_hbm,
                    rowall, colall,
                    srows0, drows0, earows0, srows1, drows1, earows1,
                    rowtail, coltail, stail, dtail, eatail,
                    sem0, sem1):
    cid, sid, wid, base = _worker_ids()

    pltpu.sync_copy(row_hbm.at[pl.ds(base, EDGES_LO)], rowall)
    pltpu.sync_copy(col_hbm.at[pl.ds(base, EDGES_LO)], colall)

    def dmas(i, sr, dr, er, sem):
        off = base + i * CHUNK
        ci = colall.at[pl.ds(i * CHUNK, CHUNK)]
        ri = rowall.at[pl.ds(i * CHUNK, CHUNK)]
        return ((s_hbm.at[ci], sr, sem), (d_hbm.at[ri], dr, sem),
                (ea_hbm.at[pl.ds(off, CHUNK)], er, sem))

    def issue(i, sr, dr, er, sem):
        for src, dst, sm in dmas(i, sr, dr, er, sem):
            pltpu.async_copy(src, dst, sm)

    def wait(i, sr, dr, er, sem):
        for src, dst, sm in dmas(i, sr, dr, er, sem):
            pltpu.make_async_copy(src, dst, sm).wait()

    def add_rows(sr, dr, er, n):
        def body(e, _):
            for cc in range(HID // LANES):
                sl = pl.ds(cc * LANES, LANES)
                er[e, sl] = er[e, sl] + sr[e, sl] + dr[e, sl]
            return 0
        lax.fori_loop(0, n, body, 0)

    def finish(i, sr, dr, er, sem):
        wait(i, sr, dr, er, sem)
        add_rows(sr, dr, er, CHUNK)
        pltpu.sync_copy(er, g_hbm.at[pl.ds(base + i * CHUNK, CHUNK)])

    issue(0, srows0, drows0, earows0, sem0)

    def body(j, _):
        i0 = 2 * j
        issue(i0 + 1, srows1, drows1, earows1, sem1)
        finish(i0, srows0, drows0, earows0, sem0)

        @pl.when(i0 + 2 < NCHUNK)
        def _nxt():
            issue(i0 + 2, srows0, drows0, earows0, sem0)

        finish(i0 + 1, srows1, drows1, earows1, sem1)
        return 0

    lax.fori_loop(0, NCHUNK // 2, body, 0)
    if NCHUNK % 2:
        finish(NCHUNK - 1, srows0, drows0, earows0, sem0)

    @pl.when(wid < 16)
    def _tail():
        off = base + NCHUNK * CHUNK
        pltpu.sync_copy(row_hbm.at[pl.ds(off, LANES)], rowtail)
        pltpu.sync_copy(col_hbm.at[pl.ds(off, LANES)], coltail)
        d1 = pltpu.async_copy(s_hbm.at[coltail], stail, sem0)
        d2 = pltpu.async_copy(d_hbm.at[rowtail], dtail, sem0)
        pltpu.sync_copy(ea_hbm.at[pl.ds(off, LANES)], eatail)
        d1.wait()
        d2.wait()
        add_rows(stail, dtail, eatail, LANES)
        pltpu.sync_copy(eatail, g_hbm.at[pl.ds(off, LANES)])

    pltpu.sync_copy(z2_hbm.at[pl.ds(0, PADROWS)],
                    g_hbm.at[pl.ds(N_EDGES + wid * PADROWS, PADROWS)])


_sc_gather = functools.partial(
    pl.kernel,
    out_type=jax.ShapeDtypeStruct((E_PAD, HID), jnp.float32),
    mesh=_MESH,
    scratch_types=[
        pltpu.VMEM((EDGES_LO,), jnp.int32),
        pltpu.VMEM((EDGES_LO,), jnp.int32),
        pltpu.VMEM((CHUNK, HID), jnp.float32),
        pltpu.VMEM((CHUNK, HID), jnp.float32),
        pltpu.VMEM((CHUNK, HID), jnp.float32),
        pltpu.VMEM((CHUNK, HID), jnp.float32),
        pltpu.VMEM((CHUNK, HID), jnp.float32),
        pltpu.VMEM((CHUNK, HID), jnp.float32),
        pltpu.VMEM((LANES,), jnp.int32),
        pltpu.VMEM((LANES,), jnp.int32),
        pltpu.VMEM((LANES, HID), jnp.float32),
        pltpu.VMEM((LANES, HID), jnp.float32),
        pltpu.VMEM((LANES, HID), jnp.float32),
        pltpu.SemaphoreType.DMA,
        pltpu.SemaphoreType.DMA,
    ],
)(_sc_gather_body)


def _sc_denom_body(l_hbm, row_hbm, pmax_hbm, ea_hbm, zcat_hbm,
                   spea_hbm,
                   pmaxbuf, rowbuf, lbuf, exbuf, catrows, earows,
                   rowbuf1, lbuf1, earows1,
                   rowtail, ltail, extail, cattail, eatail,
                   acc_sh, sem0, sem1):
    cid, sid, wid, base = _worker_ids()

    pltpu.sync_copy(pmax_hbm, pmaxbuf)
    m16 = pmaxbuf[0, pl.ds(0, LANES)]

    noff = sid * NSLICE
    pltpu.sync_copy(zcat_hbm.at[pl.ds(noff, NSLICE)],
                    acc_sh.at[pl.ds(noff, NSLICE)])
    pltpu.sync_copy(zcat_hbm.at[pl.ds(0, CHUNK)], catrows)
    pltpu.sync_copy(zcat_hbm.at[pl.ds(0, LANES)], cattail)
    plsc.subcore_barrier()

    def scale_rows(er, cat, ex, n):
        def body(e, _):
            a = ex[pl.ds(e, LANES)][0]
            cat[e, pl.ds(0, EF)] = er[e, :] * a
            cat[e, pl.ds(EF, LANES)] = jnp.full((LANES,), a, jnp.float32)
            return 0
        lax.fori_loop(0, n, body, 0)

    def dmas(i, rb, lb, er, sem):
        off = base + i * CHUNK
        return ((row_hbm.at[pl.ds(off, CHUNK)], rb, sem),
                (ea_hbm.at[pl.ds(off, CHUNK)], er, sem),
                (l_hbm.at[pl.ds(off, CHUNK)], lb, sem))

    def issue(i, rb, lb, er, sem):
        for src, dst, sm in dmas(i, rb, lb, er, sem):
            pltpu.async_copy(src, dst, sm)

    def finish(i, rb, lb, er, sem):
        for src, dst, sm in dmas(i, rb, lb, er, sem):
            pltpu.make_async_copy(src, dst, sm).wait()
        for v in range(CHUNK // LANES):
            sl = pl.ds(v * LANES, LANES)
            exbuf[sl] = jnp.exp(lb[sl] - m16)
        scale_rows(er, catrows, exbuf, CHUNK)
        pltpu.sync_copy(catrows, acc_sh.at[rb], add=True)

    issue(0, rowbuf, lbuf, earows, sem0)

    def body(j, _):
        i0 = 2 * j
        issue(i0 + 1, rowbuf1, lbuf1, earows1, sem1)
        finish(i0, rowbuf, lbuf, earows, sem0)

        @pl.when(i0 + 2 < NCHUNK)
        def _nxt():
            issue(i0 + 2, rowbuf, lbuf, earows, sem0)

        finish(i0 + 1, rowbuf1, lbuf1, earows1, sem1)
        return 0

    lax.fori_loop(0, NCHUNK // 2, body, 0)
    if NCHUNK % 2:
        finish(NCHUNK - 1, rowbuf, lbuf, earows, sem0)

    @pl.when(wid < 16)
    def _tail():
        off = base + NCHUNK * CHUNK
        pltpu.sync_copy(row_hbm.at[pl.ds(off, LANES)], rowtail)
        pltpu.sync_copy(ea_hbm.at[pl.ds(off, LANES)], eatail)
        pltpu.sync_copy(l_hbm.at[pl.ds(off, LANES)], ltail)
        extail[pl.ds(0, LANES)] = jnp.exp(ltail[...] - m16)
        scale_rows(eatail, cattail, extail, LANES)
        pltpu.sync_copy(cattail, acc_sh.at[rowtail], add=True)

    plsc.subcore_barrier()
    pltpu.sync_copy(acc_sh.at[pl.ds(noff, NSLICE)],
                    spea_hbm.at[cid, pl.ds(noff, NSLICE)])


_sc_denom = functools.partial(
    pl.kernel,
    out_type=jax.ShapeDtypeStruct((NC, N_PAD, HID), jnp.float32),
    mesh=_MESH,
    scratch_types=[
        pltpu.VMEM((8, HID), jnp.float32),
        pltpu.VMEM((CHUNK,), jnp.int32),
        pltpu.VMEM((CHUNK,), jnp.float32),
        pltpu.VMEM((CHUNK + LANES,), jnp.float32),
        pltpu.VMEM((CHUNK, HID), jnp.float32),
        pltpu.VMEM((CHUNK, EF), jnp.float32),
        pltpu.VMEM((CHUNK,), jnp.int32),
        pltpu.VMEM((CHUNK,), jnp.float32),
        pltpu.VMEM((CHUNK, EF), jnp.float32),
        pltpu.VMEM((LANES,), jnp.int32),
        pltpu.VMEM((LANES,), jnp.float32),
        pltpu.VMEM((2 * LANES,), jnp.float32),
        pltpu.VMEM((LANES, HID), jnp.float32),
        pltpu.VMEM((LANES, EF), jnp.float32),
        pltpu.VMEM_SHARED((N_PAD, HID), jnp.float32),
        pltpu.SemaphoreType.DMA,
        pltpu.SemaphoreType.DMA,
    ],
)(_sc_denom_body)


def _sc_aggm_body(l_hbm, row_hbm, col_hbm, pmax_hbm, m_hbm, z2_hbm,
                  aggm_hbm,
                  pmaxbuf, colall, rowbuf, lbuf, mrows,
                  rowbuf1, lbuf1, mrows1, exbuf,
                  rowtail, coltail, ltail, extail, mtail,
                  aggm_sh, sem0, sem1):
    cid, sid, wid, base = _worker_ids()

    pltpu.sync_copy(pmax_hbm, pmaxbuf)
    m16 = pmaxbuf[0, pl.ds(0, LANES)]
    pltpu.sync_copy(col_hbm.at[pl.ds(base, EDGES_LO)], colall)

    noff = sid * NSLICE
    pltpu.sync_copy(z2_hbm.at[pl.ds(noff, NSLICE)],
                    aggm_sh.at[pl.ds(noff, NSLICE)])
    plsc.subcore_barrier()

    def scale_rows(mr, ex, n):
        def body(e, _):
            a = ex[pl.ds(e, LANES)][0]
            for cc in range(HID // LANES):
                sl = pl.ds(cc * LANES, LANES)
                mr[e, sl] = mr[e, sl] * a
            return 0
        lax.fori_loop(0, n, body, 0)

    def dmas(i, rb, lb, mr, sem):
        off = base + i * CHUNK
        ci = colall.at[pl.ds(i * CHUNK, CHUNK)]
        return ((row_hbm.at[pl.ds(off, CHUNK)], rb, sem),
                (l_hbm.at[pl.ds(off, CHUNK)], lb, sem),
                (m_hbm.at[ci], mr, sem))

    def issue(i, rb, lb, mr, sem):
        for src, dst, sm in dmas(i, rb, lb, mr, sem):
            pltpu.async_copy(src, dst, sm)

    def finish(i, rb, lb, mr, sem):
        for src, dst, sm in dmas(i, rb, lb, mr, sem):
            pltpu.make_async_copy(src, dst, sm).wait()
        for v in range(CHUNK // LANES):
            sl = pl.ds(v * LANES, LANES)
            exbuf[sl] = jnp.exp(lb[sl] - m16)
        scale_rows(mr, exbuf, CHUNK)
        pltpu.sync_copy(mr, aggm_sh.at[rb], add=True)

    issue(0, rowbuf, lbuf, mrows, sem0)

    def body(j, _):
        i0 = 2 * j
        issue(i0 + 1, rowbuf1, lbuf1, mrows1, sem1)
        finish(i0, rowbuf, lbuf, mrows, sem0)

        @pl.when(i0 + 2 < NCHUNK)
        def _nxt():
            issue(i0 + 2, rowbuf, lbuf, mrows, sem0)

        finish(i0 + 1, rowbuf1, lbuf1, mrows1, sem1)
        return 0

    lax.fori_loop(0, NCHUNK // 2, body, 0)
    if NCHUNK % 2:
        finish(NCHUNK - 1, rowbuf, lbuf, mrows, sem0)

    @pl.when(wid < 16)
    def _tail():
        off = base + NCHUNK * CHUNK
        pltpu.sync_copy(row_hbm.at[pl.ds(off, LANES)], rowtail)
        pltpu.sync_copy(col_hbm.at[pl.ds(off, LANES)], coltail)
        d1 = pltpu.async_copy(m_hbm.at[coltail], mtail, sem0)
        pltpu.sync_copy(l_hbm.at[pl.ds(off, LANES)], ltail)
        extail[pl.ds(0, LANES)] = jnp.exp(ltail[...] - m16)
        d1.wait()
        scale_rows(mtail, extail, LANES)
        pltpu.sync_copy(mtail, aggm_sh.at[rowtail], add=True)

    plsc.subcore_barrier()
    pltpu.sync_copy(aggm_sh.at[pl.ds(noff, NSLICE)],
                    aggm_hbm.at[cid, pl.ds(noff, NSLICE)])


_sc_aggm = functools.partial(
    pl.kernel,
    out_type=jax.ShapeDtypeStruct((NC, N_PAD, HID), jnp.float32),
    mesh=_MESH,
    scratch_types=[
        pltpu.VMEM((8, HID), jnp.float32),
        pltpu.VMEM((EDGES_LO,), jnp.int32),
        pltpu.VMEM((CHUNK,), jnp.int32),
        pltpu.VMEM((CHUNK,), jnp.float32),
        pltpu.VMEM((CHUNK, HID), jnp.float32),
        pltpu.VMEM((CHUNK,), jnp.int32),
        pltpu.VMEM((CHUNK,), jnp.float32),
        pltpu.VMEM((CHUNK, HID), jnp.float32),
        pltpu.VMEM((CHUNK + LANES,), jnp.float32),
        pltpu.VMEM((LANES,), jnp.int32),
        pltpu.VMEM((LANES,), jnp.int32),
        pltpu.VMEM((LANES,), jnp.float32),
        pltpu.VMEM((2 * LANES,), jnp.float32),
        pltpu.VMEM((LANES, HID), jnp.float32),
        pltpu.VMEM_SHARED((N_PAD, HID), jnp.float32),
        pltpu.SemaphoreType.DMA,
        pltpu.SemaphoreType.DMA,
    ],
)(_sc_aggm_body)


_NBLK = 1000


def _in_proj_kern(x_ref, w_ref, b_ref, o_ref):
    o_ref[...] = jnp.maximum(
        jnp.dot(x_ref[...], w_ref[...], preferred_element_type=jnp.float32)
        + b_ref[...], 0.0)


def _tc_in_proj(x, W, b):
    n, k = x.shape
    return pl.pallas_call(
        _in_proj_kern,
        grid=(n // _NBLK,),
        in_specs=[pl.BlockSpec((_NBLK, k), lambda i: (i, 0)),
                  pl.BlockSpec((k, HID), lambda i: (0, 0)),
                  pl.BlockSpec((HID,), lambda i: (0,))],
        out_specs=pl.BlockSpec((_NBLK, HID), lambda i: (i, 0)),
        out_shape=jax.ShapeDtypeStruct((n, HID), jnp.float32),
    )(x, W, b)


def _node4_kern(h_ref, w_ref, b_ref, s_ref, d_ref, m_ref, z_ref):
    r = jnp.dot(h_ref[...], w_ref[...],
                preferred_element_type=jnp.float32) + b_ref[...]
    s_ref[...] = r[:, 0 * HID:1 * HID]
    d_ref[...] = r[:, 1 * HID:2 * HID]
    m_ref[...] = r[:, 2 * HID:3 * HID]
    z_ref[...] = r[:, 3 * HID:4 * HID]


def _tc_node4(h, Wcat, bcat):
    n = h.shape[0]
    out = jax.ShapeDtypeStruct((n, HID), jnp.float32)
    return pl.pallas_call(
        _node4_kern,
        grid=(n // _NBLK,),
        in_specs=[pl.BlockSpec((_NBLK, HID), lambda i: (i, 0)),
                  pl.BlockSpec((HID, 4 * HID), lambda i: (0, 0)),
                  pl.BlockSpec((4 * HID,), lambda i: (0,))],
        out_specs=[pl.BlockSpec((_NBLK, HID), lambda i: (i, 0))] * 4,
        out_shape=(out, out, out, out),
    )(h, Wcat, bcat)


_EBLK = 2000


def _edge_proj_kern(ea_ref, w1_ref, b1_ref, w2_ref, b2_ref, o1_ref, o2_ref):
    ea = ea_ref[...]
    o1_ref[...] = jnp.dot(ea, w1_ref[...],
                          preferred_element_type=jnp.float32) + b1_ref[...]
    o2_ref[...] = jnp.dot(ea, w2_ref[...],
                          preferred_element_type=jnp.float32) + b2_ref[...]


def _tc_edge_proj2(ea, W1, b1, W2, b2):
    n = ea.shape[0]
    out = jax.ShapeDtypeStruct((n, HID), jnp.float32)
    return pl.pallas_call(
        _edge_proj_kern,
        grid=(n // _EBLK,),
        in_specs=[pl.BlockSpec((_EBLK, EF), lambda i: (i, 0)),
                  pl.BlockSpec((EF, HID), lambda i: (0, 0)),
                  pl.BlockSpec((HID,), lambda i: (0,)),
                  pl.BlockSpec((EF, HID), lambda i: (0, 0)),
                  pl.BlockSpec((HID,), lambda i: (0,))],
        out_specs=[pl.BlockSpec((_EBLK, HID), lambda i: (i, 0))] * 2,
        out_shape=(out, out),
    )(ea, W1, b1, W2, b2)


def _logit_kern(g_ref, w_ref, l_ref, pm_ref):
    r = jnp.dot(jnp.maximum(g_ref[...], 0.0), w_ref[...],
                preferred_element_type=jnp.float32)
    l_ref[...] = r
    bm = jnp.full((8, HID), jnp.max(r), jnp.float32)

    @pl.when(pl.program_id(0) == 0)
    def _init():
        pm_ref[...] = bm

    @pl.when(pl.program_id(0) != 0)
    def _acc():
        pm_ref[...] = jnp.maximum(pm_ref[...], bm)


def _tc_logit(G, wdot):
    return pl.pallas_call(
        _logit_kern,
        grid=(NBLK_L,),
        in_specs=[pl.BlockSpec((EBLK_L, HID), lambda i: (i, 0)),
                  pl.BlockSpec((HID, 1), lambda i: (0, 0))],
        out_specs=[pl.BlockSpec((EBLK_L, 1), lambda i: (i, 0)),
                   pl.BlockSpec((8, HID), lambda i: (0, 0))],
        out_shape=(jax.ShapeDtypeStruct((E_PAD, 1), jnp.float32),
                   jax.ShapeDtypeStruct((8, HID), jnp.float32)),
    )(G, wdot)


def _combine_kern(am_ref, spea_ref, wme_ref, bme_ref, z_ref, ai_ref,
                  o_ref):
    aggm = am_ref[0] + am_ref[1]
    cat = spea_ref[0] + spea_ref[1]
    aggea = cat[:, :EF]
    s = cat[:, EF]
    inv = 1.0 / (s + 1e-16)
    t = s * inv
    aggr = ((aggm
             + jnp.dot(aggea, wme_ref[...], preferred_element_type=jnp.float32))
            * inv[:, None]
            + t[:, None] * bme_ref[...][None, :])
    o_ref[...] = jnp.maximum(aggr + z_ref[...] + ai_ref[...], 0.0)


def _tc_combine(aggm, spea, Wme, bme, Z, atom_in):
    n = Z.shape[0]
    return pl.pallas_call(
        _combine_kern,
        grid=(n // _NBLK,),
        in_specs=[pl.BlockSpec((NC, _NBLK, HID), lambda i: (0, i, 0)),
                  pl.BlockSpec((NC, _NBLK, HID), lambda i: (0, i, 0)),
                  pl.BlockSpec((EF, HID), lambda i: (0, 0)),
                  pl.BlockSpec((HID,), lambda i: (0,)),
                  pl.BlockSpec((_NBLK, HID), lambda i: (i, 0)),
                  pl.BlockSpec((_NBLK, HID), lambda i: (i, 0))],
        out_specs=pl.BlockSpec((_NBLK, HID), lambda i: (i, 0)),
        out_shape=jax.ShapeDtypeStruct((n, HID), jnp.float32),
    )(aggm, spea, Wme, bme, Z, atom_in)


def kernel(x, edge_attr, edge_index, params):
    row = edge_index[0]
    col = edge_index[1]
    W_in, b_in = params['atom_inp']

    atom_input = _tc_in_proj(x, W_in, b_in)
    atom_h = atom_input

    zeros2 = jnp.zeros((N_PAD, HID), jnp.float32)

    lps = params['layers']
    EAs = _tc_edge_proj2(edge_attr,
                         lps[0]['attn_edg'][0], lps[0]['attn_edg'][1],
                         lps[1]['attn_edg'][0], lps[1]['attn_edg'][1])

    for li, lp in enumerate(lps):
        Wcat = jnp.concatenate([lp['attn_src'][0], lp['attn_dst'][0],
                                lp['msg_dst'][0], lp['wgt_n'][0]], axis=1)
        bcat = jnp.concatenate([lp['attn_src'][1], lp['attn_dst'][1],
                                lp['msg_dst'][1], lp['wgt_n'][1]])
        S, D, M, Z = _tc_node4(atom_h, Wcat, bcat)
        EA = EAs[li]

        G = _sc_gather(S, D, EA, row, col, zeros2)
        l2d, pmax = _tc_logit(G, lp['attn_dot'][0])
        l = l2d.reshape(E_PAD)
        spea = _sc_denom(l, row, pmax, edge_attr, zeros2)
        aggm = _sc_aggm(l, row, col, pmax, M, zeros2)

        atom_h = _tc_combine(aggm, spea, lp['msg_edg'][0],
                             lp['msg_edg'][1], Z, atom_input)

    return atom_h


# speedup vs baseline: 1.0126x; 1.0126x over previous
import functools

import jax
import jax.numpy as jnp
from jax import lax
from jax.experimental import pallas as pl
from jax.experimental.pallas import tpu as pltpu
from jax.experimental.pallas import tpu_sc as plsc

N_NODES = 10000
N_PAD = 10240
N_EDGES = 160000
E_PAD = 163840
HID = 128
EF = 16
NC = 2
NS = 16
NW = NC * NS
LANES = 16
NSLICE = N_PAD // NS
PADROWS = (E_PAD - N_EDGES) // NW

EDGES_HI = 5008
EDGES_LO = 4992
CHUNK = 96
NCHUNK = 52
NBLK_L = 40
EBLK_L = E_PAD // NBLK_L

_MESH = plsc.VectorSubcoreMesh(
    core_axis_name="c", subcore_axis_name="s", num_cores=NC, num_subcores=NS)


def _worker_ids():
    cid = lax.axis_index("c")
    sid = lax.axis_index("s")
    wid = sid * NC + cid
    base = jnp.where(wid < 16, wid * EDGES_HI,
                     16 * EDGES_HI + (wid - 16) * EDGES_LO)
    return cid, sid, wid, base


def _sc_gather_body(s_hbm, d_hbm, ea_hbm, row_hbm, col_hbm, z2_hbm,
                    g_hbm,
                    rowall, colall,
                    srows0, drows0, earows0, srows1, drows1, earows1,
                    rowtail, coltail, stail, dtail, eatail,
                    sem0, sem1):
    cid, sid, wid, base = _worker_ids()

    pltpu.sync_copy(row_hbm.at[pl.ds(base, EDGES_LO)], rowall)
    pltpu.sync_copy(col_hbm.at[pl.ds(base, EDGES_LO)], colall)

    def dmas(i, sr, dr, er, sem):
        off = base + i * CHUNK
        ci = colall.at[pl.ds(i * CHUNK, CHUNK)]
        ri = rowall.at[pl.ds(i * CHUNK, CHUNK)]
        return ((s_hbm.at[ci], sr, sem), (d_hbm.at[ri], dr, sem),
                (ea_hbm.at[pl.ds(off, CHUNK)], er, sem))

    def issue(i, sr, dr, er, sem):
        for src, dst, sm in dmas(i, sr, dr, er, sem):
            pltpu.async_copy(src, dst, sm)

    def wait(i, sr, dr, er, sem):
        for src, dst, sm in dmas(i, sr, dr, er, sem):
            pltpu.make_async_copy(src, dst, sm).wait()

    def add_rows(sr, dr, er, n):
        def body(e, _):
            for cc in range(HID // LANES):
                sl = pl.ds(cc * LANES, LANES)
                er[e, sl] = er[e, sl] + sr[e, sl] + dr[e, sl]
            return 0
        lax.fori_loop(0, n, body, 0)

    def finish(i, sr, dr, er, sem):
        wait(i, sr, dr, er, sem)
        add_rows(sr, dr, er, CHUNK)
        pltpu.sync_copy(er, g_hbm.at[pl.ds(base + i * CHUNK, CHUNK)])

    issue(0, srows0, drows0, earows0, sem0)

    def body(j, _):
        i0 = 2 * j
        issue(i0 + 1, srows1, drows1, earows1, sem1)
        finish(i0, srows0, drows0, earows0, sem0)

        @pl.when(i0 + 2 < NCHUNK)
        def _nxt():
            issue(i0 + 2, srows0, drows0, earows0, sem0)

        finish(i0 + 1, srows1, drows1, earows1, sem1)
        return 0

    lax.fori_loop(0, NCHUNK // 2, body, 0)
    if NCHUNK % 2:
        finish(NCHUNK - 1, srows0, drows0, earows0, sem0)

    @pl.when(wid < 16)
    def _tail():
        off = base + NCHUNK * CHUNK
        pltpu.sync_copy(row_hbm.at[pl.ds(off, LANES)], rowtail)
        pltpu.sync_copy(col_hbm.at[pl.ds(off, LANES)], coltail)
        d1 = pltpu.async_copy(s_hbm.at[coltail], stail, sem0)
        d2 = pltpu.async_copy(d_hbm.at[rowtail], dtail, sem0)
        pltpu.sync_copy(ea_hbm.at[pl.ds(off, LANES)], eatail)
        d1.wait()
        d2.wait()
        add_rows(stail, dtail, eatail, LANES)
        pltpu.sync_copy(eatail, g_hbm.at[pl.ds(off, LANES)])

    pltpu.sync_copy(z2_hbm.at[pl.ds(0, PADROWS)],
                    g_hbm.at[pl.ds(N_EDGES + wid * PADROWS, PADROWS)])


_sc_gather = functools.partial(
    pl.kernel,
    out_type=jax.ShapeDtypeStruct((E_PAD, HID), jnp.float32),
    mesh=_MESH,
    scratch_types=[
        pltpu.VMEM((EDGES_LO,), jnp.int32),
        pltpu.VMEM((EDGES_LO,), jnp.int32),
        pltpu.VMEM((CHUNK, HID), jnp.float32),
        pltpu.VMEM((CHUNK, HID), jnp.float32),
        pltpu.VMEM((CHUNK, HID), jnp.float32),
        pltpu.VMEM((CHUNK, HID), jnp.float32),
        pltpu.VMEM((CHUNK, HID), jnp.float32),
        pltpu.VMEM((CHUNK, HID), jnp.float32),
        pltpu.VMEM((LANES,), jnp.int32),
        pltpu.VMEM((LANES,), jnp.int32),
        pltpu.VMEM((LANES, HID), jnp.float32),
        pltpu.VMEM((LANES, HID), jnp.float32),
        pltpu.VMEM((LANES, HID), jnp.float32),
        pltpu.SemaphoreType.DMA,
        pltpu.SemaphoreType.DMA,
    ],
)(_sc_gather_body)


def _sc_denom_body(l_hbm, row_hbm, pmax_hbm, ea_hbm, zcat_hbm,
                   spea_hbm,
                   pmaxbuf, rowbuf, lbuf, exbuf, catrows, earows,
                   rowbuf1, lbuf1, earows1,
                   rowtail, ltail, extail, cattail, eatail,
                   acc_sh, sem0, sem1):
    cid, sid, wid, base = _worker_ids()

    pltpu.sync_copy(pmax_hbm, pmaxbuf)
    m16 = pmaxbuf[0, pl.ds(0, LANES)]

    noff = sid * NSLICE
    pltpu.sync_copy(zcat_hbm.at[pl.ds(noff, NSLICE)],
                    acc_sh.at[pl.ds(noff, NSLICE)])
    pltpu.sync_copy(zcat_hbm.at[pl.ds(0, CHUNK)], catrows)
    pltpu.sync_copy(zcat_hbm.at[pl.ds(0, LANES)], cattail)
    plsc.subcore_barrier()

    def scale_rows(er, cat, ex, n):
        def body(e, _):
            a = ex[pl.ds(e, LANES)][0]
            cat[e, pl.ds(0, EF)] = er[e, :] * a
            cat[e, pl.ds(EF, LANES)] = jnp.full((LANES,), a, jnp.float32)
            return 0
        lax.fori_loop(0, n, body, 0)

    def dmas(i, rb, lb, er, sem):
        off = base + i * CHUNK
        return ((row_hbm.at[pl.ds(off, CHUNK)], rb, sem),
                (ea_hbm.at[pl.ds(off, CHUNK)], er, sem),
                (l_hbm.at[pl.ds(off, CHUNK)], lb, sem))

    def issue(i, rb, lb, er, sem):
        for src, dst, sm in dmas(i, rb, lb, er, sem):
            pltpu.async_copy(src, dst, sm)

    def finish(i, rb, lb, er, sem):
        for src, dst, sm in dmas(i, rb, lb, er, sem):
            pltpu.make_async_copy(src, dst, sm).wait()
        for v in range(CHUNK // LANES):
            sl = pl.ds(v * LANES, LANES)
            exbuf[sl] = jnp.exp(lb[sl] - m16)
        scale_rows(er, catrows, exbuf, CHUNK)
        pltpu.sync_copy(catrows, acc_sh.at[rb], add=True)

    issue(0, rowbuf, lbuf, earows, sem0)

    def body(j, _):
        i0 = 2 * j
        issue(i0 + 1, rowbuf1, lbuf1, earows1, sem1)
        finish(i0, rowbuf, lbuf, earows, sem0)

        @pl.when(i0 + 2 < NCHUNK)
        def _nxt():
            issue(i0 + 2, rowbuf, lbuf, earows, sem0)

        finish(i0 + 1, rowbuf1, lbuf1, earows1, sem1)
        return 0

    lax.fori_loop(0, NCHUNK // 2, body, 0)
    if NCHUNK % 2:
        finish(NCHUNK - 1, rowbuf, lbuf, earows, sem0)

    @pl.when(wid < 16)
    def _tail():
        off = base + NCHUNK * CHUNK
        pltpu.sync_copy(row_hbm.at[pl.ds(off, LANES)], rowtail)
        pltpu.sync_copy(ea_hbm.at[pl.ds(off, LANES)], eatail)
        pltpu.sync_copy(l_hbm.at[pl.ds(off, LANES)], ltail)
        extail[pl.ds(0, LANES)] = jnp.exp(ltail[...] - m16)
        scale_rows(eatail, cattail, extail, LANES)
        pltpu.sync_copy(cattail, acc_sh.at[rowtail], add=True)

    plsc.subcore_barrier()
    pltpu.sync_copy(acc_sh.at[pl.ds(noff, NSLICE)],
                    spea_hbm.at[cid, pl.ds(noff, NSLICE)])


_sc_denom = functools.partial(
    pl.kernel,
    out_type=jax.ShapeDtypeStruct((NC, N_PAD, HID), jnp.float32),
    mesh=_MESH,
    scratch_types=[
        pltpu.VMEM((8, HID), jnp.float32),
        pltpu.VMEM((CHUNK,), jnp.int32),
        pltpu.VMEM((CHUNK,), jnp.float32),
        pltpu.VMEM((CHUNK + LANES,), jnp.float32),
        pltpu.VMEM((CHUNK, HID), jnp.float32),
        pltpu.VMEM((CHUNK, EF), jnp.float32),
        pltpu.VMEM((CHUNK,), jnp.int32),
        pltpu.VMEM((CHUNK,), jnp.float32),
        pltpu.VMEM((CHUNK, EF), jnp.float32),
        pltpu.VMEM((LANES,), jnp.int32),
        pltpu.VMEM((LANES,), jnp.float32),
        pltpu.VMEM((2 * LANES,), jnp.float32),
        pltpu.VMEM((LANES, HID), jnp.float32),
        pltpu.VMEM((LANES, EF), jnp.float32),
        pltpu.VMEM_SHARED((N_PAD, HID), jnp.float32),
        pltpu.SemaphoreType.DMA,
        pltpu.SemaphoreType.DMA,
    ],
)(_sc_denom_body)


def _sc_aggm_body(l_hbm, row_hbm, col_hbm, pmax_hbm, m_hbm, z2_hbm,
                  aggm_hbm,
                  pmaxbuf, colall, rowbuf, lbuf, mrows,
                  rowbuf1, lbuf1, mrows1, exbuf,
                  rowtail, coltail, ltail, extail, mtail,
                  aggm_sh, sem0, sem1):
    cid, sid, wid, base = _worker_ids()

    pltpu.sync_copy(pmax_hbm, pmaxbuf)
    m16 = pmaxbuf[0, pl.ds(0, LANES)]
    pltpu.sync_copy(col_hbm.at[pl.ds(base, EDGES_LO)], colall)

    noff = sid * NSLICE
    pltpu.sync_copy(z2_hbm.at[pl.ds(noff, NSLICE)],
                    aggm_sh.at[pl.ds(noff, NSLICE)])
    plsc.subcore_barrier()

    def scale_rows(mr, ex, n):
        def body(e, _):
            a = ex[pl.ds(e, LANES)][0]
            for cc in range(HID // LANES):
                sl = pl.ds(cc * LANES, LANES)
                mr[e, sl] = mr[e, sl] * a
            return 0
        lax.fori_loop(0, n, body, 0)

    def dmas(i, rb, lb, mr, sem):
        off = base + i * CHUNK
        ci = colall.at[pl.ds(i * CHUNK, CHUNK)]
        return ((row_hbm.at[pl.ds(off, CHUNK)], rb, sem),
                (l_hbm.at[pl.ds(off, CHUNK)], lb, sem),
                (m_hbm.at[ci], mr, sem))

    def issue(i, rb, lb, mr, sem):
        for src, dst, sm in dmas(i, rb, lb, mr, sem):
            pltpu.async_copy(src, dst, sm)

    def finish(i, rb, lb, mr, sem):
        for src, dst, sm in dmas(i, rb, lb, mr, sem):
            pltpu.make_async_copy(src, dst, sm).wait()
        for v in range(CHUNK // LANES):
            sl = pl.ds(v * LANES, LANES)
            exbuf[sl] = jnp.exp(lb[sl] - m16)
        scale_rows(mr, exbuf, CHUNK)
        pltpu.sync_copy(mr, aggm_sh.at[rb], add=True)

    issue(0, rowbuf, lbuf, mrows, sem0)

    def body(j, _):
        i0 = 2 * j
        issue(i0 + 1, rowbuf1, lbuf1, mrows1, sem1)
        finish(i0, rowbuf, lbuf, mrows, sem0)

        @pl.when(i0 + 2 < NCHUNK)
        def _nxt():
            issue(i0 + 2, rowbuf, lbuf, mrows, sem0)

        finish(i0 + 1, rowbuf1, lbuf1, mrows1, sem1)
        return 0

    lax.fori_loop(0, NCHUNK // 2, body, 0)
    if NCHUNK % 2:
        finish(NCHUNK - 1, rowbuf, lbuf, mrows, sem0)

    @pl.when(wid < 16)
    def _tail():
        off = base + NCHUNK * CHUNK
        pltpu.sync_copy(row_hbm.at[pl.ds(off, LANES)], rowtail)
        pltpu.sync_copy(col_hbm.at[pl.ds(off, LANES)], coltail)
        d1 = pltpu.async_copy(m_hbm.at[coltail], mtail, sem0)
        pltpu.sync_copy(l_hbm.at[pl.ds(off, LANES)], ltail)
        extail[pl.ds(0, LANES)] = jnp.exp(ltail[...] - m16)
        d1.wait()
        scale_rows(mtail, extail, LANES)
        pltpu.sync_copy(mtail, aggm_sh.at[rowtail], add=True)

    plsc.subcore_barrier()
    pltpu.sync_copy(aggm_sh.at[pl.ds(noff, NSLICE)],
                    aggm_hbm.at[cid, pl.ds(noff, NSLICE)])


_sc_aggm = functools.partial(
    pl.kernel,
    out_type=jax.ShapeDtypeStruct((NC, N_PAD, HID), jnp.float32),
    mesh=_MESH,
    scratch_types=[
        pltpu.VMEM((8, HID), jnp.float32),
        pltpu.VMEM((EDGES_LO,), jnp.int32),
        pltpu.VMEM((CHUNK,), jnp.int32),
        pltpu.VMEM((CHUNK,), jnp.float32),
        pltpu.VMEM((CHUNK, HID), jnp.float32),
        pltpu.VMEM((CHUNK,), jnp.int32),
        pltpu.VMEM((CHUNK,), jnp.float32),
        pltpu.VMEM((CHUNK, HID), jnp.float32),
        pltpu.VMEM((CHUNK + LANES,), jnp.float32),
        pltpu.VMEM((LANES,), jnp.int32),
        pltpu.VMEM((LANES,), jnp.int32),
        pltpu.VMEM((LANES,), jnp.float32),
        pltpu.VMEM((2 * LANES,), jnp.float32),
        pltpu.VMEM((LANES, HID), jnp.float32),
        pltpu.VMEM_SHARED((N_PAD, HID), jnp.float32),
        pltpu.SemaphoreType.DMA,
        pltpu.SemaphoreType.DMA,
    ],
)(_sc_aggm_body)


_NBLK = 1000


def _in_proj_kern(x_ref, w_ref, b_ref, o_ref):
    o_ref[...] = jnp.maximum(
        jnp.dot(x_ref[...], w_ref[...], preferred_element_type=jnp.float32)
        + b_ref[...], 0.0)


def _tc_in_proj(x, W, b):
    n, k = x.shape
    return pl.pallas_call(
        _in_proj_kern,
        grid=(n // _NBLK,),
        in_specs=[pl.BlockSpec((_NBLK, k), lambda i: (i, 0)),
                  pl.BlockSpec((k, HID), lambda i: (0, 0)),
                  pl.BlockSpec((HID,), lambda i: (0,))],
        out_specs=pl.BlockSpec((_NBLK, HID), lambda i: (i, 0)),
        out_shape=jax.ShapeDtypeStruct((n, HID), jnp.float32),
    )(x, W, b)


def _node4_kern(h_ref, w_ref, b_ref, s_ref, d_ref, m_ref, z_ref):
    r = jnp.dot(h_ref[...], w_ref[...],
                preferred_element_type=jnp.float32) + b_ref[...]
    s_ref[...] = r[:, 0 * HID:1 * HID]
    d_ref[...] = r[:, 1 * HID:2 * HID]
    m_ref[...] = r[:, 2 * HID:3 * HID]
    z_ref[...] = r[:, 3 * HID:4 * HID]


def _tc_node4(h, Wcat, bcat):
    n = h.shape[0]
    out = jax.ShapeDtypeStruct((n, HID), jnp.float32)
    return pl.pallas_call(
        _node4_kern,
        grid=(n // _NBLK,),
        in_specs=[pl.BlockSpec((_NBLK, HID), lambda i: (i, 0)),
                  pl.BlockSpec((HID, 4 * HID), lambda i: (0, 0)),
                  pl.BlockSpec((4 * HID,), lambda i: (0,))],
        out_specs=[pl.BlockSpec((_NBLK, HID), lambda i: (i, 0))] * 4,
        out_shape=(out, out, out, out),
    )(h, Wcat, bcat)


_EBLK = 2000


def _edge_proj_kern(ea_ref, w_ref, b_ref, o_ref):
    o_ref[...] = jnp.dot(ea_ref[...], w_ref[...],
                         preferred_element_type=jnp.float32) + b_ref[...]


def _tc_edge_proj(ea, W, b):
    n = ea.shape[0]
    return pl.pallas_call(
        _edge_proj_kern,
        grid=(n // _EBLK,),
        in_specs=[pl.BlockSpec((_EBLK, EF), lambda i: (i, 0)),
                  pl.BlockSpec((EF, HID), lambda i: (0, 0)),
                  pl.BlockSpec((HID,), lambda i: (0,))],
        out_specs=pl.BlockSpec((_EBLK, HID), lambda i: (i, 0)),
        out_shape=jax.ShapeDtypeStruct((n, HID), jnp.float32),
    )(ea, W, b)


def _logit_kern(g_ref, w_ref, l_ref, pm_ref):
    r = jnp.dot(jnp.maximum(g_ref[...], 0.0), w_ref[...],
                preferred_element_type=jnp.float32)
    l_ref[...] = r
    bm = jnp.full((8, HID), jnp.max(r), jnp.float32)

    @pl.when(pl.program_id(0) == 0)
    def _init():
        pm_ref[...] = bm

    @pl.when(pl.program_id(0) != 0)
    def _acc():
        pm_ref[...] = jnp.maximum(pm_ref[...], bm)


def _tc_logit(G, wdot):
    return pl.pallas_call(
        _logit_kern,
        grid=(NBLK_L,),
        in_specs=[pl.BlockSpec((EBLK_L, HID), lambda i: (i, 0)),
                  pl.BlockSpec((HID, 1), lambda i: (0, 0))],
        out_specs=[pl.BlockSpec((EBLK_L, 1), lambda i: (i, 0)),
                   pl.BlockSpec((8, HID), lambda i: (0, 0))],
        out_shape=(jax.ShapeDtypeStruct((E_PAD, 1), jnp.float32),
                   jax.ShapeDtypeStruct((8, HID), jnp.float32)),
    )(G, wdot)


def _combine_kern(am_ref, spea_ref, wme_ref, bme_ref, z_ref, ai_ref,
                  o_ref):
    aggm = am_ref[0] + am_ref[1]
    cat = spea_ref[0] + spea_ref[1]
    aggea = cat[:, :EF]
    s = cat[:, EF]
    inv = 1.0 / (s + 1e-16)
    t = s * inv
    aggr = ((aggm
             + jnp.dot(aggea, wme_ref[...], preferred_element_type=jnp.float32))
            * inv[:, None]
            + t[:, None] * bme_ref[...][None, :])
    o_ref[...] = jnp.maximum(aggr + z_ref[...] + ai_ref[...], 0.0)


def _tc_combine(aggm, spea, Wme, bme, Z, atom_in):
    n = Z.shape[0]
    return pl.pallas_call(
        _combine_kern,
        grid=(n // _NBLK,),
        in_specs=[pl.BlockSpec((NC, _NBLK, HID), lambda i: (0, i, 0)),
                  pl.BlockSpec((NC, _NBLK, HID), lambda i: (0, i, 0)),
                  pl.BlockSpec((EF, HID), lambda i: (0, 0)),
                  pl.BlockSpec((HID,), lambda i: (0,)),
                  pl.BlockSpec((_NBLK, HID), lambda i: (i, 0)),
                  pl.BlockSpec((_NBLK, HID), lambda i: (i, 0))],
        out_specs=pl.BlockSpec((_NBLK, HID), lambda i: (i, 0)),
        out_shape=jax.ShapeDtypeStruct((n, HID), jnp.float32),
    )(aggm, spea, Wme, bme, Z, atom_in)


def kernel(x, edge_attr, edge_index, params):
    row = edge_index[0]
    col = edge_index[1]
    W_in, b_in = params['atom_inp']

    atom_input = _tc_in_proj(x, W_in, b_in)
    atom_h = atom_input

    zeros2 = jnp.zeros((N_PAD, HID), jnp.float32)

    for lp in params['layers']:
        Wcat = jnp.concatenate([lp['attn_src'][0], lp['attn_dst'][0],
                                lp['msg_dst'][0], lp['wgt_n'][0]], axis=1)
        bcat = jnp.concatenate([lp['attn_src'][1], lp['attn_dst'][1],
                                lp['msg_dst'][1], lp['wgt_n'][1]])
        S, D, M, Z = _tc_node4(atom_h, Wcat, bcat)
        EA = _tc_edge_proj(edge_attr, lp['attn_edg'][0], lp['attn_edg'][1])

        G = _sc_gather(S, D, EA, row, col, zeros2)
        l2d, pmax = _tc_logit(G, lp['attn_dot'][0])
        l = l2d.reshape(E_PAD)
        spea = _sc_denom(l, row, pmax, edge_attr, zeros2)
        aggm = _sc_aggm(l, row, col, pmax, M, zeros2)

        atom_h = _tc_combine(aggm, spea, lp['msg_edg'][0],
                             lp['msg_edg'][1], Z, atom_input)

    return atom_h


# speedup vs baseline: 1.0144x; 1.0017x over previous
import functools

import jax
import jax.numpy as jnp
from jax import lax
from jax.experimental import pallas as pl
from jax.experimental.pallas import tpu as pltpu
from jax.experimental.pallas import tpu_sc as plsc

N_NODES = 10000
N_PAD = 10240
N_EDGES = 160000
E_PAD = 163840
HID = 128
EF = 16
NC = 2
NS = 16
NW = NC * NS
LANES = 16
NSLICE = N_PAD // NS
PADROWS = (E_PAD - N_EDGES) // NW

EDGES_HI = 5008
EDGES_LO = 4992
CHUNK = 96
NCHUNK = 52
NBLK_L = 40
EBLK_L = E_PAD // NBLK_L

_MESH = plsc.VectorSubcoreMesh(
    core_axis_name="c", subcore_axis_name="s", num_cores=NC, num_subcores=NS)


def _worker_ids():
    cid = lax.axis_index("c")
    sid = lax.axis_index("s")
    wid = sid * NC + cid
    base = jnp.where(wid < 16, wid * EDGES_HI,
                     16 * EDGES_HI + (wid - 16) * EDGES_LO)
    return cid, sid, wid, base


def _sc_gather_body(s_hbm, d_hbm, ea_hbm, row_hbm, col_hbm, z2_hbm,
                    g_hbm,
                    rowall, colall,
                    srows0, drows0, earows0, srows1, drows1, earows1,
                    obuf0, obuf1,
                    rowtail, coltail, stail, dtail, eatail,
                    sem0, sem1):
    cid, sid, wid, base = _worker_ids()

    pltpu.sync_copy(row_hbm.at[pl.ds(base, EDGES_LO)], rowall)
    pltpu.sync_copy(col_hbm.at[pl.ds(base, EDGES_LO)], colall)

    def dmas(i, sr, dr, er, sem):
        off = base + i * CHUNK
        ci = colall.at[pl.ds(i * CHUNK, CHUNK)]
        ri = rowall.at[pl.ds(i * CHUNK, CHUNK)]
        return ((s_hbm.at[ci], sr, sem), (d_hbm.at[ri], dr, sem),
                (ea_hbm.at[pl.ds(off, CHUNK)], er, sem))

    def issue(i, sr, dr, er, sem):
        for src, dst, sm in dmas(i, sr, dr, er, sem):
            pltpu.async_copy(src, dst, sm)

    def wait(i, sr, dr, er, sem):
        for src, dst, sm in dmas(i, sr, dr, er, sem):
            pltpu.make_async_copy(src, dst, sm).wait()

    def add_rows_to(sr, dr, er, ob, n):
        def body(e, _):
            for cc in range(HID // LANES):
                sl = pl.ds(cc * LANES, LANES)
                ob[e, sl] = er[e, sl] + sr[e, sl] + dr[e, sl]
            return 0
        lax.fori_loop(0, n, body, 0)

    def finish(i, sr, dr, er, ob, sem):
        wait(i, sr, dr, er, sem)
        add_rows_to(sr, dr, er, ob, CHUNK)
        pltpu.sync_copy(ob, g_hbm.at[pl.ds(base + i * CHUNK, CHUNK)])

    issue(0, srows0, drows0, earows0, sem0)

    def body(j, _):
        i0 = 2 * j
        issue(i0 + 1, srows1, drows1, earows1, sem1)
        finish(i0, srows0, drows0, earows0, obuf0, sem0)

        @pl.when(i0 + 2 < NCHUNK)
        def _nxt():
            issue(i0 + 2, srows0, drows0, earows0, sem0)

        finish(i0 + 1, srows1, drows1, earows1, obuf1, sem1)
        return 0

    lax.fori_loop(0, NCHUNK // 2, body, 0)

    @pl.when(wid < 16)
    def _tail():
        off = base + NCHUNK * CHUNK
        pltpu.sync_copy(row_hbm.at[pl.ds(off, LANES)], rowtail)
        pltpu.sync_copy(col_hbm.at[pl.ds(off, LANES)], coltail)
        d1 = pltpu.async_copy(s_hbm.at[coltail], stail, sem0)
        d2 = pltpu.async_copy(d_hbm.at[rowtail], dtail, sem0)
        pltpu.sync_copy(ea_hbm.at[pl.ds(off, LANES)], eatail)
        d1.wait()
        d2.wait()
        add_rows_to(stail, dtail, eatail, eatail, LANES)
        pltpu.sync_copy(eatail, g_hbm.at[pl.ds(off, LANES)])

    pltpu.sync_copy(z2_hbm.at[pl.ds(0, PADROWS)],
                    g_hbm.at[pl.ds(N_EDGES + wid * PADROWS, PADROWS)])


_sc_gather = functools.partial(
    pl.kernel,
    out_type=jax.ShapeDtypeStruct((E_PAD, HID), jnp.float32),
    mesh=_MESH,
    scratch_types=[
        pltpu.VMEM((EDGES_LO,), jnp.int32),
        pltpu.VMEM((EDGES_LO,), jnp.int32),
        pltpu.VMEM((CHUNK, HID), jnp.float32),
        pltpu.VMEM((CHUNK, HID), jnp.float32),
        pltpu.VMEM((CHUNK, HID), jnp.float32),
        pltpu.VMEM((CHUNK, HID), jnp.float32),
        pltpu.VMEM((CHUNK, HID), jnp.float32),
        pltpu.VMEM((CHUNK, HID), jnp.float32),
        pltpu.VMEM((CHUNK, HID), jnp.float32),
        pltpu.VMEM((CHUNK, HID), jnp.float32),
        pltpu.VMEM((LANES,), jnp.int32),
        pltpu.VMEM((LANES,), jnp.int32),
        pltpu.VMEM((LANES, HID), jnp.float32),
        pltpu.VMEM((LANES, HID), jnp.float32),
        pltpu.VMEM((LANES, HID), jnp.float32),
        pltpu.SemaphoreType.DMA,
        pltpu.SemaphoreType.DMA,
    ],
)(_sc_gather_body)


def _sc_denom_body(l_hbm, row_hbm, pmax_hbm, ea_hbm, zcat_hbm,
                   spea_hbm,
                   pmaxbuf, rowbuf, lbuf, exbuf, catrows, earows,
                   rowbuf1, lbuf1, earows1,
                   rowtail, ltail, extail, cattail, eatail,
                   acc_sh, sem0, sem1):
    cid, sid, wid, base = _worker_ids()

    pltpu.sync_copy(pmax_hbm, pmaxbuf)
    m16 = pmaxbuf[0, pl.ds(0, LANES)]

    noff = sid * NSLICE
    pltpu.sync_copy(zcat_hbm.at[pl.ds(noff, NSLICE)],
                    acc_sh.at[pl.ds(noff, NSLICE)])
    pltpu.sync_copy(zcat_hbm.at[pl.ds(0, CHUNK)], catrows)
    pltpu.sync_copy(zcat_hbm.at[pl.ds(0, LANES)], cattail)
    plsc.subcore_barrier()

    def scale_rows(er, cat, ex, n):
        def body(e, _):
            a = ex[pl.ds(e, LANES)][0]
            cat[e, pl.ds(0, EF)] = er[e, :] * a
            cat[e, pl.ds(EF, LANES)] = jnp.full((LANES,), a, jnp.float32)
            return 0
        lax.fori_loop(0, n, body, 0)

    def dmas(i, rb, lb, er, sem):
        off = base + i * CHUNK
        return ((row_hbm.at[pl.ds(off, CHUNK)], rb, sem),
                (ea_hbm.at[pl.ds(off, CHUNK)], er, sem),
                (l_hbm.at[pl.ds(off, CHUNK)], lb, sem))

    def issue(i, rb, lb, er, sem):
        for src, dst, sm in dmas(i, rb, lb, er, sem):
            pltpu.async_copy(src, dst, sm)

    def finish(i, rb, lb, er, sem):
        for src, dst, sm in dmas(i, rb, lb, er, sem):
            pltpu.make_async_copy(src, dst, sm).wait()
        for v in range(CHUNK // LANES):
            sl = pl.ds(v * LANES, LANES)
            exbuf[sl] = jnp.exp(lb[sl] - m16)
        scale_rows(er, catrows, exbuf, CHUNK)
        pltpu.sync_copy(catrows, acc_sh.at[rb], add=True)

    issue(0, rowbuf, lbuf, earows, sem0)

    def body(j, _):
        i0 = 2 * j
        issue(i0 + 1, rowbuf1, lbuf1, earows1, sem1)
        finish(i0, rowbuf, lbuf, earows, sem0)

        @pl.when(i0 + 2 < NCHUNK)
        def _nxt():
            issue(i0 + 2, rowbuf, lbuf, earows, sem0)

        finish(i0 + 1, rowbuf1, lbuf1, earows1, sem1)
        return 0

    lax.fori_loop(0, NCHUNK // 2, body, 0)
    if NCHUNK % 2:
        finish(NCHUNK - 1, rowbuf, lbuf, earows, sem0)

    @pl.when(wid < 16)
    def _tail():
        off = base + NCHUNK * CHUNK
        pltpu.sync_copy(row_hbm.at[pl.ds(off, LANES)], rowtail)
        pltpu.sync_copy(ea_hbm.at[pl.ds(off, LANES)], eatail)
        pltpu.sync_copy(l_hbm.at[pl.ds(off, LANES)], ltail)
        extail[pl.ds(0, LANES)] = jnp.exp(ltail[...] - m16)
        scale_rows(eatail, cattail, extail, LANES)
        pltpu.sync_copy(cattail, acc_sh.at[rowtail], add=True)

    plsc.subcore_barrier()
    pltpu.sync_copy(acc_sh.at[pl.ds(noff, NSLICE)],
                    spea_hbm.at[cid, pl.ds(noff, NSLICE)])


_sc_denom = functools.partial(
    pl.kernel,
    out_type=jax.ShapeDtypeStruct((NC, N_PAD, HID), jnp.float32),
    mesh=_MESH,
    scratch_types=[
        pltpu.VMEM((8, HID), jnp.float32),
        pltpu.VMEM((CHUNK,), jnp.int32),
        pltpu.VMEM((CHUNK,), jnp.float32),
        pltpu.VMEM((CHUNK + LANES,), jnp.float32),
        pltpu.VMEM((CHUNK, HID), jnp.float32),
        pltpu.VMEM((CHUNK, EF), jnp.float32),
        pltpu.VMEM((CHUNK,), jnp.int32),
        pltpu.VMEM((CHUNK,), jnp.float32),
        pltpu.VMEM((CHUNK, EF), jnp.float32),
        pltpu.VMEM((LANES,), jnp.int32),
        pltpu.VMEM((LANES,), jnp.float32),
        pltpu.VMEM((2 * LANES,), jnp.float32),
        pltpu.VMEM((LANES, HID), jnp.float32),
        pltpu.VMEM((LANES, EF), jnp.float32),
        pltpu.VMEM_SHARED((N_PAD, HID), jnp.float32),
        pltpu.SemaphoreType.DMA,
        pltpu.SemaphoreType.DMA,
    ],
)(_sc_denom_body)


def _sc_aggm_body(l_hbm, row_hbm, col_hbm, pmax_hbm, m_hbm, z2_hbm,
                  aggm_hbm,
                  pmaxbuf, colall, rowbuf, lbuf, mrows,
                  rowbuf1, lbuf1, mrows1, exbuf,
                  rowtail, coltail, ltail, extail, mtail,
                  aggm_sh, sem0, sem1):
    cid, sid, wid, base = _worker_ids()

    pltpu.sync_copy(pmax_hbm, pmaxbuf)
    m16 = pmaxbuf[0, pl.ds(0, LANES)]
    pltpu.sync_copy(col_hbm.at[pl.ds(base, EDGES_LO)], colall)

    noff = sid * NSLICE
    pltpu.sync_copy(z2_hbm.at[pl.ds(noff, NSLICE)],
                    aggm_sh.at[pl.ds(noff, NSLICE)])
    plsc.subcore_barrier()

    def scale_rows(mr, ex, n):
        def body(e, _):
            a = ex[pl.ds(e, LANES)][0]
            for cc in range(HID // LANES):
                sl = pl.ds(cc * LANES, LANES)
                mr[e, sl] = mr[e, sl] * a
            return 0
        lax.fori_loop(0, n, body, 0)

    def dmas(i, rb, lb, mr, sem):
        off = base + i * CHUNK
        ci = colall.at[pl.ds(i * CHUNK, CHUNK)]
        return ((row_hbm.at[pl.ds(off, CHUNK)], rb, sem),
                (l_hbm.at[pl.ds(off, CHUNK)], lb, sem),
                (m_hbm.at[ci], mr, sem))

    def issue(i, rb, lb, mr, sem):
        for src, dst, sm in dmas(i, rb, lb, mr, sem):
            pltpu.async_copy(src, dst, sm)

    def finish(i, rb, lb, mr, sem):
        for src, dst, sm in dmas(i, rb, lb, mr, sem):
            pltpu.make_async_copy(src, dst, sm).wait()
        for v in range(CHUNK // LANES):
            sl = pl.ds(v * LANES, LANES)
            exbuf[sl] = jnp.exp(lb[sl] - m16)
        scale_rows(mr, exbuf, CHUNK)
        pltpu.sync_copy(mr, aggm_sh.at[rb], add=True)

    issue(0, rowbuf, lbuf, mrows, sem0)

    def body(j, _):
        i0 = 2 * j
        issue(i0 + 1, rowbuf1, lbuf1, mrows1, sem1)
        finish(i0, rowbuf, lbuf, mrows, sem0)

        @pl.when(i0 + 2 < NCHUNK)
        def _nxt():
            issue(i0 + 2, rowbuf, lbuf, mrows, sem0)

        finish(i0 + 1, rowbuf1, lbuf1, mrows1, sem1)
        return 0

    lax.fori_loop(0, NCHUNK // 2, body, 0)
    if NCHUNK % 2:
        finish(NCHUNK - 1, rowbuf, lbuf, mrows, sem0)

    @pl.when(wid < 16)
    def _tail():
        off = base + NCHUNK * CHUNK
        pltpu.sync_copy(row_hbm.at[pl.ds(off, LANES)], rowtail)
        pltpu.sync_copy(col_hbm.at[pl.ds(off, LANES)], coltail)
        d1 = pltpu.async_copy(m_hbm.at[coltail], mtail, sem0)
        pltpu.sync_copy(l_hbm.at[pl.ds(off, LANES)], ltail)
        extail[pl.ds(0, LANES)] = jnp.exp(ltail[...] - m16)
        d1.wait()
        scale_rows(mtail, extail, LANES)
        pltpu.sync_copy(mtail, aggm_sh.at[rowtail], add=True)

    plsc.subcore_barrier()
    pltpu.sync_copy(aggm_sh.at[pl.ds(noff, NSLICE)],
                    aggm_hbm.at[cid, pl.ds(noff, NSLICE)])


_sc_aggm = functools.partial(
    pl.kernel,
    out_type=jax.ShapeDtypeStruct((NC, N_PAD, HID), jnp.float32),
    mesh=_MESH,
    scratch_types=[
        pltpu.VMEM((8, HID), jnp.float32),
        pltpu.VMEM((EDGES_LO,), jnp.int32),
        pltpu.VMEM((CHUNK,), jnp.int32),
        pltpu.VMEM((CHUNK,), jnp.float32),
        pltpu.VMEM((CHUNK, HID), jnp.float32),
        pltpu.VMEM((CHUNK,), jnp.int32),
        pltpu.VMEM((CHUNK,), jnp.float32),
        pltpu.VMEM((CHUNK, HID), jnp.float32),
        pltpu.VMEM((CHUNK + LANES,), jnp.float32),
        pltpu.VMEM((LANES,), jnp.int32),
        pltpu.VMEM((LANES,), jnp.int32),
        pltpu.VMEM((LANES,), jnp.float32),
        pltpu.VMEM((2 * LANES,), jnp.float32),
        pltpu.VMEM((LANES, HID), jnp.float32),
        pltpu.VMEM_SHARED((N_PAD, HID), jnp.float32),
        pltpu.SemaphoreType.DMA,
        pltpu.SemaphoreType.DMA,
    ],
)(_sc_aggm_body)


_NBLK = 1000


def _in_proj_kern(x_ref, w_ref, b_ref, o_ref):
    o_ref[...] = jnp.maximum(
        jnp.dot(x_ref[...], w_ref[...], preferred_element_type=jnp.float32)
        + b_ref[...], 0.0)


def _tc_in_proj(x, W, b):
    n, k = x.shape
    return pl.pallas_call(
        _in_proj_kern,
        grid=(n // _NBLK,),
        in_specs=[pl.BlockSpec((_NBLK, k), lambda i: (i, 0)),
                  pl.BlockSpec((k, HID), lambda i: (0, 0)),
                  pl.BlockSpec((HID,), lambda i: (0,))],
        out_specs=pl.BlockSpec((_NBLK, HID), lambda i: (i, 0)),
        out_shape=jax.ShapeDtypeStruct((n, HID), jnp.float32),
    )(x, W, b)


def _node4_kern(h_ref, w_ref, b_ref, s_ref, d_ref, m_ref, z_ref):
    r = jnp.dot(h_ref[...], w_ref[...],
                preferred_element_type=jnp.float32) + b_ref[...]
    s_ref[...] = r[:, 0 * HID:1 * HID]
    d_ref[...] = r[:, 1 * HID:2 * HID]
    m_ref[...] = r[:, 2 * HID:3 * HID]
    z_ref[...] = r[:, 3 * HID:4 * HID]


def _tc_node4(h, Wcat, bcat):
    n = h.shape[0]
    out = jax.ShapeDtypeStruct((n, HID), jnp.float32)
    return pl.pallas_call(
        _node4_kern,
        grid=(n // _NBLK,),
        in_specs=[pl.BlockSpec((_NBLK, HID), lambda i: (i, 0)),
                  pl.BlockSpec((HID, 4 * HID), lambda i: (0, 0)),
                  pl.BlockSpec((4 * HID,), lambda i: (0,))],
        out_specs=[pl.BlockSpec((_NBLK, HID), lambda i: (i, 0))] * 4,
        out_shape=(out, out, out, out),
    )(h, Wcat, bcat)


_EBLK = 2000


def _edge_proj_kern(ea_ref, w_ref, b_ref, o_ref):
    o_ref[...] = jnp.dot(ea_ref[...], w_ref[...],
                         preferred_element_type=jnp.float32) + b_ref[...]


def _tc_edge_proj(ea, W, b):
    n = ea.shape[0]
    return pl.pallas_call(
        _edge_proj_kern,
        grid=(n // _EBLK,),
        in_specs=[pl.BlockSpec((_EBLK, EF), lambda i: (i, 0)),
                  pl.BlockSpec((EF, HID), lambda i: (0, 0)),
                  pl.BlockSpec((HID,), lambda i: (0,))],
        out_specs=pl.BlockSpec((_EBLK, HID), lambda i: (i, 0)),
        out_shape=jax.ShapeDtypeStruct((n, HID), jnp.float32),
    )(ea, W, b)


def _logit_kern(g_ref, w_ref, l_ref, pm_ref):
    r = jnp.dot(jnp.maximum(g_ref[...], 0.0), w_ref[...],
                preferred_element_type=jnp.float32)
    l_ref[...] = r
    bm = jnp.full((8, HID), jnp.max(r), jnp.float32)

    @pl.when(pl.program_id(0) == 0)
    def _init():
        pm_ref[...] = bm

    @pl.when(pl.program_id(0) != 0)
    def _acc():
        pm_ref[...] = jnp.maximum(pm_ref[...], bm)


def _tc_logit(G, wdot):
    return pl.pallas_call(
        _logit_kern,
        grid=(NBLK_L,),
        in_specs=[pl.BlockSpec((EBLK_L, HID), lambda i: (i, 0)),
                  pl.BlockSpec((HID, 1), lambda i: (0, 0))],
        out_specs=[pl.BlockSpec((EBLK_L, 1), lambda i: (i, 0)),
                   pl.BlockSpec((8, HID), lambda i: (0, 0))],
        out_shape=(jax.ShapeDtypeStruct((E_PAD, 1), jnp.float32),
                   jax.ShapeDtypeStruct((8, HID), jnp.float32)),
    )(G, wdot)


def _combine_kern(am_ref, spea_ref, wme_ref, bme_ref, z_ref, ai_ref,
                  o_ref):
    aggm = am_ref[0] + am_ref[1]
    cat = spea_ref[0] + spea_ref[1]
    aggea = cat[:, :EF]
    s = cat[:, EF]
    inv = 1.0 / (s + 1e-16)
    t = s * inv
    aggr = ((aggm
             + jnp.dot(aggea, wme_ref[...], preferred_element_type=jnp.float32))
            * inv[:, None]
            + t[:, None] * bme_ref[...][None, :])
    o_ref[...] = jnp.maximum(aggr + z_ref[...] + ai_ref[...], 0.0)


def _tc_combine(aggm, spea, Wme, bme, Z, atom_in):
    n = Z.shape[0]
    return pl.pallas_call(
        _combine_kern,
        grid=(n // _NBLK,),
        in_specs=[pl.BlockSpec((NC, _NBLK, HID), lambda i: (0, i, 0)),
                  pl.BlockSpec((NC, _NBLK, HID), lambda i: (0, i, 0)),
                  pl.BlockSpec((EF, HID), lambda i: (0, 0)),
                  pl.BlockSpec((HID,), lambda i: (0,)),
                  pl.BlockSpec((_NBLK, HID), lambda i: (i, 0)),
                  pl.BlockSpec((_NBLK, HID), lambda i: (i, 0))],
        out_specs=pl.BlockSpec((_NBLK, HID), lambda i: (i, 0)),
        out_shape=jax.ShapeDtypeStruct((n, HID), jnp.float32),
    )(aggm, spea, Wme, bme, Z, atom_in)


def kernel(x, edge_attr, edge_index, params):
    row = edge_index[0]
    col = edge_index[1]
    W_in, b_in = params['atom_inp']

    atom_input = _tc_in_proj(x, W_in, b_in)
    atom_h = atom_input

    zeros2 = jnp.zeros((N_PAD, HID), jnp.float32)

    for lp in params['layers']:
        Wcat = jnp.concatenate([lp['attn_src'][0], lp['attn_dst'][0],
                                lp['msg_dst'][0], lp['wgt_n'][0]], axis=1)
        bcat = jnp.concatenate([lp['attn_src'][1], lp['attn_dst'][1],
                                lp['msg_dst'][1], lp['wgt_n'][1]])
        S, D, M, Z = _tc_node4(atom_h, Wcat, bcat)
        EA = _tc_edge_proj(edge_attr, lp['attn_edg'][0], lp['attn_edg'][1])

        G = _sc_gather(S, D, EA, row, col, zeros2)
        l2d, pmax = _tc_logit(G, lp['attn_dot'][0])
        l = l2d.reshape(E_PAD)
        spea = _sc_denom(l, row, pmax, edge_attr, zeros2)
        aggm = _sc_aggm(l, row, col, pmax, M, zeros2)

        atom_h = _tc_combine(aggm, spea, lp['msg_edg'][0],
                             lp['msg_edg'][1], Z, atom_input)

    return atom_h
